# baseline (speedup 1.0000x reference)
import jax
import jax.numpy as jnp
from jax import lax
from jax.experimental import pallas as pl
from jax.experimental.pallas import tpu as pltpu

T = 2048
H = 1024
E = 16
I = 256
TOPK = 2
NG = 4
TG = 2
RSF = 2.5

EC = 2
NC = E // EC
CI = EC * I

_NEG = float(jnp.finfo(jnp.float32).min)


def _silu(x):
    return x * jax.nn.sigmoid(x)


def _dotT(a, b):
    return lax.dot_general(a, b, (((1,), (1,)), ((), ())),
                           preferred_element_type=jnp.float32)


def _routing(xb, xlo, Wgh, Wgl, bias2):
    logits = (_dotT(xb, Wgh) + (_dotT(xlo, Wgh) + _dotT(xb, Wgl)))
    scores = jax.nn.sigmoid(logits)
    sfc = scores + bias2
    lane = lax.broadcasted_iota(jnp.int32, (T, E), 1)
    gid = lane // (E // NG)
    gsc = jnp.zeros((T, E), jnp.float32)
    for g in range(NG):
        gs = jnp.where(gid == g, sfc, _NEG)
        m1 = jnp.max(gs, axis=1, keepdims=True)
        m2 = jnp.max(jnp.where(gs >= m1, _NEG, gs), axis=1, keepdims=True)
        gsc = jnp.where(gid == g, m1 + m2, gsc)
    gmax1 = jnp.max(gsc, axis=1, keepdims=True)
    mask1 = gsc >= gmax1
    gmax2 = jnp.max(jnp.where(mask1, _NEG, gsc), axis=1, keepdims=True)
    smask = mask1 | (gsc >= gmax2)
    masked = jnp.where(smask, sfc, _NEG)
    t1 = jnp.max(masked, axis=1, keepdims=True)
    em1 = masked >= t1
    t2 = jnp.max(jnp.where(em1, _NEG, masked), axis=1, keepdims=True)
    sel = em1 | (masked >= t2)
    w = jnp.where(sel, scores, 0.0)
    denom = jnp.sum(w, axis=1, keepdims=True) + 1e-20
    return w / denom * RSF


def _hc_chunk(xb, w1, w3, comb, chunk, hc_slot_ref):
    lane = lax.broadcasted_iota(jnp.int32, (T, E), 1)
    for k in range(EC):
        sl = slice(k * I, (k + 1) * I)
        h1 = _dotT(xb, w1[sl].astype(jnp.bfloat16))
        h3 = _dotT(xb, w3[sl].astype(jnp.bfloat16))
        col = jnp.sum(jnp.where(lane == EC * chunk + k, comb, 0.0),
                      axis=1, keepdims=True)
        hc_slot_ref[:, sl] = (_silu(h1) * h3 * col).astype(jnp.bfloat16)


def _moe_body(xb_ref, xlo_ref, Wgh_ref, Wgl_ref, b_ref,
              W1_ref, W3_ref, W2_ref,
              W1p_ref, W3p_ref, Ws1_ref, Ws3_ref, Ws2_ref,
              out_ref, comb_ref, hc_ref, w2t_ref):
    c = pl.program_id(0)

    @pl.when(c == 0)
    def _():
        xb0 = xb_ref[...]
        h = (_silu(_dotT(xb0, Ws1_ref[...].astype(jnp.bfloat16)))
             * _dotT(xb0, Ws3_ref[...].astype(jnp.bfloat16))
             ).astype(jnp.bfloat16)
        ws2b = Ws2_ref[...].astype(jnp.bfloat16)
        for tb in range(4):
            st = slice(tb * (T // 4), (tb + 1) * (T // 4))
            out_ref[st, :] = _dotT(h[st, :], ws2b)
        comb_ref[...] = _routing(xb0, xlo_ref[...], Wgh_ref[...],
                                 Wgl_ref[...], b_ref[...])
        _hc_chunk(xb0, W1p_ref[...], W3p_ref[...], comb_ref[...], 0,
                  hc_ref.at[0])

    xb = xb_ref[...]
    comb = comb_ref[...]
    nxt = jnp.minimum(c + 1, NC - 1)
    _hc_chunk(xb, W1_ref[...], W3_ref[...], comb, nxt,
              hc_ref.at[(c + 1) % 2])
    w2t_ref[...] = (jnp.transpose(W2_ref[...], (0, 2, 1))
                    .reshape(CI, H).astype(jnp.bfloat16))
    w2t = w2t_ref[...]
    for tb in range(4):
        st = slice(tb * (T // 4), (tb + 1) * (T // 4))
        out_ref[st, :] += jnp.dot(hc_ref[c % 2, st, :], w2t,
                                  preferred_element_type=jnp.float32)


def kernel(hidden_states, Wg, bias, W1, W3, W2, Ws1, Ws3, Ws2):
    bias2 = bias.reshape(1, E)
    xb = hidden_states.astype(jnp.bfloat16)
    xlo = (hidden_states - xb.astype(jnp.float32)).astype(jnp.bfloat16)
    Wgh = Wg.astype(jnp.bfloat16)
    Wgl = (Wg - Wgh.astype(jnp.float32)).astype(jnp.bfloat16)
    W1r = W1.reshape(E * I, H)
    W3r = W3.reshape(E * I, H)
    W1p = W1r[:CI].astype(jnp.bfloat16)
    W3p = W3r[:CI].astype(jnp.bfloat16)
    out, _ = pl.pallas_call(
        _moe_body,
        grid=(NC,),
        in_specs=[
            pl.BlockSpec((T, H), lambda c: (0, 0)),
            pl.BlockSpec((T, H), lambda c: (0, 0)),
            pl.BlockSpec((E, H), lambda c: (0, 0)),
            pl.BlockSpec((E, H), lambda c: (0, 0)),
            pl.BlockSpec((1, E), lambda c: (0, 0)),
            pl.BlockSpec((CI, H), lambda c: (jnp.minimum(c + 1, NC - 1), 0)),
            pl.BlockSpec((CI, H), lambda c: (jnp.minimum(c + 1, NC - 1), 0)),
            pl.BlockSpec((EC, H, I), lambda c: (c, 0, 0)),
            pl.BlockSpec((CI, H), lambda c: (0, 0)),
            pl.BlockSpec((CI, H), lambda c: (0, 0)),
            pl.BlockSpec((I, H), lambda c: (0, 0)),
            pl.BlockSpec((I, H), lambda c: (0, 0)),
            pl.BlockSpec((H, I), lambda c: (0, 0)),
        ],
        out_specs=[
            pl.BlockSpec((T, H), lambda c: (0, 0)),
            pl.BlockSpec((T, E), lambda c: (0, 0)),
        ],
        out_shape=[
            jax.ShapeDtypeStruct((T, H), jnp.float32),
            jax.ShapeDtypeStruct((T, E), jnp.float32),
        ],
        scratch_shapes=[pltpu.VMEM((2, T, CI), jnp.bfloat16),
                        pltpu.VMEM((CI, H), jnp.bfloat16)],
        compiler_params=pltpu.CompilerParams(
            dimension_semantics=("arbitrary",),
            vmem_limit_bytes=100 * 1024 * 1024),
    )(xb, xlo, Wgh, Wgl, bias2, W1r, W3r, W2, W1p, W3p, Ws1, Ws3, Ws2)
    return out


# speedup vs baseline: 1.3141x; 1.3141x over previous
import jax
import jax.numpy as jnp
from jax import lax
from jax.experimental import pallas as pl
from jax.experimental.pallas import tpu as pltpu

T = 2048
H = 1024
E = 16
I = 256
TOPK = 2
NG = 4
TG = 2
RSF = 2.5

EC = 2
NC = E // EC
CI = EC * I

_NEG = float(jnp.finfo(jnp.float32).min)


def _silu(x):
    return x * jax.nn.sigmoid(x)


def _dotT(a, b):
    return lax.dot_general(a, b, (((1,), (1,)), ((), ())),
                           preferred_element_type=jnp.float32)


def _routing(x, Wg, bias2):
    logits = _dotT(x, Wg)
    scores = jax.nn.sigmoid(logits)
    sfc = scores + bias2
    lane = lax.broadcasted_iota(jnp.int32, (T, E), 1)
    gid = lane // (E // NG)
    gsc = jnp.zeros((T, E), jnp.float32)
    for g in range(NG):
        gs = jnp.where(gid == g, sfc, _NEG)
        m1 = jnp.max(gs, axis=1, keepdims=True)
        m2 = jnp.max(jnp.where(gs >= m1, _NEG, gs), axis=1, keepdims=True)
        gsc = jnp.where(gid == g, m1 + m2, gsc)
    gmax1 = jnp.max(gsc, axis=1, keepdims=True)
    mask1 = gsc >= gmax1
    gmax2 = jnp.max(jnp.where(mask1, _NEG, gsc), axis=1, keepdims=True)
    smask = mask1 | (gsc >= gmax2)
    masked = jnp.where(smask, sfc, _NEG)
    t1 = jnp.max(masked, axis=1, keepdims=True)
    em1 = masked >= t1
    t2 = jnp.max(jnp.where(em1, _NEG, masked), axis=1, keepdims=True)
    sel = em1 | (masked >= t2)
    w = jnp.where(sel, scores, 0.0)
    denom = jnp.sum(w, axis=1, keepdims=True) + 1e-20
    return w / denom * RSF


def _moe_body(x_ref, Wg_ref, b_ref, W1_ref, W3_ref, W2_ref,
              Ws1_ref, Ws3_ref, Ws2_ref, out_ref, comb_ref, hc_ref, xb_ref):
    c = pl.program_id(0)

    @pl.when(c == 0)
    def _():
        x = x_ref[...]
        xb0 = x.astype(jnp.bfloat16)
        xb_ref[...] = xb0
        h = (_silu(_dotT(xb0, Ws1_ref[...].astype(jnp.bfloat16)))
             * _dotT(xb0, Ws3_ref[...].astype(jnp.bfloat16)))
        out_ref[...] = _dotT(h.astype(jnp.bfloat16),
                             Ws2_ref[...].astype(jnp.bfloat16))
        comb_ref[...] = _routing(x, Wg_ref[...], b_ref[...])

    xb = xb_ref[...]
    comb = comb_ref[...]
    lane = lax.broadcasted_iota(jnp.int32, (T, E), 1)
    for k in range(EC):
        sl = slice(k * I, (k + 1) * I)
        h1 = _dotT(xb, W1_ref[sl, :].astype(jnp.bfloat16))
        h3 = _dotT(xb, W3_ref[sl, :].astype(jnp.bfloat16))
        col = jnp.sum(jnp.where(lane == EC * c + k, comb, 0.0),
                      axis=1, keepdims=True)
        hc_ref[:, sl] = (_silu(h1) * h3 * col).astype(jnp.bfloat16)
    w2t = jnp.transpose(W2_ref[...], (0, 2, 1)).reshape(CI, H)
    out_ref[...] += jnp.dot(hc_ref[...], w2t.astype(jnp.bfloat16),
                            preferred_element_type=jnp.float32)


def kernel(hidden_states, Wg, bias, W1, W3, W2, Ws1, Ws3, Ws2):
    bias2 = bias.reshape(1, E)
    W1r = W1.reshape(E * I, H)
    W3r = W3.reshape(E * I, H)
    out, _ = pl.pallas_call(
        _moe_body,
        grid=(NC,),
        in_specs=[
            pl.BlockSpec((T, H), lambda c: (0, 0)),
            pl.BlockSpec((E, H), lambda c: (0, 0)),
            pl.BlockSpec((1, E), lambda c: (0, 0)),
            pl.BlockSpec((CI, H), lambda c: (c, 0)),
            pl.BlockSpec((CI, H), lambda c: (c, 0)),
            pl.BlockSpec((EC, H, I), lambda c: (c, 0, 0)),
            pl.BlockSpec((I, H), lambda c: (0, 0)),
            pl.BlockSpec((I, H), lambda c: (0, 0)),
            pl.BlockSpec((H, I), lambda c: (0, 0)),
        ],
        out_specs=[
            pl.BlockSpec((T, H), lambda c: (0, 0)),
            pl.BlockSpec((T, E), lambda c: (0, 0)),
        ],
        out_shape=[
            jax.ShapeDtypeStruct((T, H), jnp.float32),
            jax.ShapeDtypeStruct((T, E), jnp.float32),
        ],
        scratch_shapes=[pltpu.VMEM((T, CI), jnp.bfloat16),
                        pltpu.VMEM((T, H), jnp.bfloat16)],
        compiler_params=pltpu.CompilerParams(
            dimension_semantics=("arbitrary",),
            vmem_limit_bytes=100 * 1024 * 1024),
    )(hidden_states, Wg, bias2, W1r, W3r, W2, Ws1, Ws3, Ws2)
    return out
